# Initial kernel scaffold; baseline (speedup 1.0000x reference)
import jax
import jax.numpy as jnp
from jax.experimental import pallas as pl


def kernel(hidden_states, W):
    raise NotImplementedError("write your pallas kernel here")


# speedup vs baseline: 6.2413x; 6.2413x over previous
import functools

import jax
import jax.numpy as jnp
from jax.experimental import pallas as pl

NUM_EXPERTS = 64
HIDDEN = 4096
TOKENS = 8192
TOKEN_BLOCK = 512


def _router_body(x_ref, w_ref, logits_ref, probs_ref, ent_ref):
    j = pl.program_id(0)
    x = x_ref[...]
    w = w_ref[...]
    logits = jax.lax.dot_general(
        x, w, (((1,), (1,)), ((), ())), preferred_element_type=jnp.float32
    )
    logits_ref[...] = logits
    m = jnp.max(logits, axis=-1, keepdims=True)
    unnorm = jnp.exp(logits - m)
    p = unnorm / jnp.sum(unnorm, axis=-1, keepdims=True)
    probs_ref[...] = p
    ent = -jnp.sum(p * jnp.log(p + 1e-8))

    @pl.when(j == 0)
    def _():
        ent_ref[...] = jnp.zeros_like(ent_ref)

    ent_ref[...] += jnp.reshape(ent, (1, 1))


def _routing_body(p_ref, sel_ref, w_ref, var_ref):
    tokens, experts = p_ref.shape
    k = min(tokens // experts, tokens)
    p = p_ref[...]
    pb = jax.lax.bitcast_convert_type(p, jnp.int32)

    tbits = jnp.zeros((1, experts), jnp.int32)
    for b in range(30, -1, -1):
        cand = tbits | jnp.int32(1 << b)
        cnt = jnp.sum((pb >= cand).astype(jnp.int32), axis=0, keepdims=True)
        tbits = jnp.where(cnt >= k, cand, tbits)

    m = jnp.sum((pb > tbits).astype(jnp.int32), axis=0, keepdims=True)
    r = k - m
    tie = pb == tbits
    idx = jax.lax.broadcasted_iota(jnp.int32, (tokens, 1), 0)

    istar = jnp.zeros((1, experts), jnp.int32)
    for b in range(12, -1, -1):
        cand = istar + jnp.int32(1 << b)
        cnt = jnp.sum((tie & (idx < cand)).astype(jnp.int32), axis=0, keepdims=True)
        istar = jnp.where(cnt < r, cand, istar)

    sel_mask = (pb > tbits) | (tie & (idx <= istar))
    eio = jax.lax.broadcasted_iota(jnp.int32, (1, experts), 1)
    e_best = jnp.max(jnp.where(sel_mask, eio, -1), axis=1, keepdims=True)
    w_sel = jnp.max(
        jnp.where(sel_mask & (eio == e_best), p, 0.0), axis=1, keepdims=True
    )
    sel = jnp.maximum(e_best, 0)
    sel_ref[...] = sel
    w_ref[...] = w_sel

    counts = jnp.sum((sel == eio).astype(jnp.float32), axis=0, keepdims=True)
    load = counts / jnp.float32(tokens)
    mu = jnp.sum(load) / jnp.float32(experts)
    var = jnp.sum((load - mu) ** 2) / jnp.float32(experts - 1)
    var_ref[...] = jnp.reshape(var, (1, 1))


@functools.partial(jax.jit, static_argnames=())
def kernel(hidden_states, W):
    tokens, hidden = hidden_states.shape
    experts = W.shape[0]
    nblk = tokens // TOKEN_BLOCK

    logits, probs, ent_sum = pl.pallas_call(
        _router_body,
        grid=(nblk,),
        in_specs=[
            pl.BlockSpec((TOKEN_BLOCK, hidden), lambda j: (j, 0)),
            pl.BlockSpec((experts, hidden), lambda j: (0, 0)),
        ],
        out_specs=[
            pl.BlockSpec((TOKEN_BLOCK, experts), lambda j: (j, 0)),
            pl.BlockSpec((TOKEN_BLOCK, experts), lambda j: (j, 0)),
            pl.BlockSpec((1, 1), lambda j: (0, 0)),
        ],
        out_shape=[
            jax.ShapeDtypeStruct((tokens, experts), jnp.float32),
            jax.ShapeDtypeStruct((tokens, experts), jnp.float32),
            jax.ShapeDtypeStruct((1, 1), jnp.float32),
        ],
    )(hidden_states, W)

    sel, wts, var = pl.pallas_call(
        _routing_body,
        out_shape=[
            jax.ShapeDtypeStruct((tokens, 1), jnp.int32),
            jax.ShapeDtypeStruct((tokens, 1), jnp.float32),
            jax.ShapeDtypeStruct((1, 1), jnp.float32),
        ],
    )(probs)

    entropy = ent_sum[0, 0] / jnp.float32(tokens)
    return (logits, sel, wts, var[0, 0], entropy)


# speedup vs baseline: 7.1761x; 1.1498x over previous
import functools

import jax
import jax.numpy as jnp
from jax.experimental import pallas as pl
from jax.experimental.pallas import tpu as pltpu

TOKEN_BLOCK = 512


def _dup(v):
    return jnp.concatenate([v, v], axis=1)


def _routing(pw, tokens, experts):
    half = tokens // 2
    k = min(tokens // experts, tokens)
    pb = jax.lax.bitcast_convert_type(pw, jnp.int32)

    def fold(c):
        return c[:, :experts] + c[:, experts:]

    tbits = jnp.zeros((1, experts), jnp.int32)
    for b in range(29, -1, -1):
        cand = tbits | jnp.int32(1 << b)
        cnt = fold(jnp.sum((pb >= _dup(cand)).astype(jnp.int32), axis=0,
                           keepdims=True))
        tbits = jnp.where(cnt >= k, cand, tbits)

    t128 = _dup(tbits)
    gt = pb > t128
    tie = pb == t128
    cnt_gt = fold(jnp.sum(gt.astype(jnp.int32), axis=0, keepdims=True))
    cnt_tie = fold(jnp.sum(tie.astype(jnp.int32), axis=0, keepdims=True))
    r = k - cnt_gt

    laneio = jax.lax.broadcasted_iota(jnp.int32, (half, 2 * experts), 1)
    hi_half = laneio >= experts
    idxw = (jax.lax.broadcasted_iota(jnp.int32, (half, 2 * experts), 0)
            + jnp.where(hi_half, half, 0))

    def tie_free(_):
        return jnp.full((1, experts), tokens, jnp.int32)

    def with_ties(_):
        istar = jnp.zeros((1, experts), jnp.int32)
        for b in range(12, -1, -1):
            cand = istar + jnp.int32(1 << b)
            c = fold(jnp.sum((tie & (idxw < _dup(cand))).astype(jnp.int32),
                             axis=0, keepdims=True))
            istar = jnp.where(c < r, cand, istar)
        return istar

    any_tie = jnp.max(cnt_gt + cnt_tie) > k
    istar = jax.lax.cond(any_tie, with_ties, tie_free, 0)

    sel_mask = gt | (tie & (idxw <= _dup(istar)))

    eio = jnp.where(hi_half, laneio - experts, laneio)
    neg = jnp.int32(-1)
    e_lo = jnp.max(jnp.where(sel_mask & ~hi_half, eio, neg), axis=1,
                   keepdims=True)
    e_hi = jnp.max(jnp.where(sel_mask & hi_half, eio, neg), axis=1,
                   keepdims=True)
    w_lo = jnp.max(jnp.where(sel_mask & ~hi_half & (eio == e_lo), pw, 0.0),
                   axis=1, keepdims=True)
    w_hi = jnp.max(jnp.where(sel_mask & hi_half & (eio == e_hi), pw, 0.0),
                   axis=1, keepdims=True)
    sel_lo = jnp.maximum(e_lo, 0)
    sel_hi = jnp.maximum(e_hi, 0)

    selw = jnp.where(hi_half, sel_hi, sel_lo)
    counts = fold(jnp.sum((selw == eio).astype(jnp.float32), axis=0,
                          keepdims=True).astype(jnp.int32)).astype(jnp.float32)
    load = counts / jnp.float32(tokens)
    mu = jnp.sum(load) / jnp.float32(experts)
    var = jnp.sum((load - mu) ** 2) / jnp.float32(experts - 1)
    return sel_lo, sel_hi, w_lo, w_hi, var


def _fused_body(x_ref, w_ref, logits_ref, sel_ref, w_out_ref, var_ref,
                ent_ref, probs_ref):
    j = pl.program_id(0)
    nblk = pl.num_programs(0)
    x = x_ref[...]
    w = w_ref[...]
    logits = jax.lax.dot_general(
        x, w, (((1,), (1,)), ((), ())), preferred_element_type=jnp.float32
    )
    logits_ref[...] = logits
    m = jnp.max(logits, axis=-1, keepdims=True)
    unnorm = jnp.exp(logits - m)
    p = unnorm / jnp.sum(unnorm, axis=-1, keepdims=True)
    blk = logits.shape[0]
    half_blocks = nblk // 2
    row = jnp.where(j < half_blocks, j, j - half_blocks) * blk

    @pl.when(j < half_blocks)
    def _():
        probs_ref[pl.ds(row, blk), 0:64] = p

    @pl.when(j >= half_blocks)
    def _():
        probs_ref[pl.ds(row, blk), 64:128] = p

    ent = -jnp.sum(p * jnp.log(p + 1e-8))

    @pl.when(j == 0)
    def _():
        ent_ref[...] = jnp.zeros_like(ent_ref)

    ent_ref[...] += jnp.reshape(ent, (1, 1))

    @pl.when(j == nblk - 1)
    def _():
        half, twoe = probs_ref.shape
        tokens = half * 2
        experts = twoe // 2
        sel_lo, sel_hi, w_lo, w_hi, var = _routing(
            probs_ref[...], tokens, experts)
        sel_ref[0:half, :] = sel_lo
        sel_ref[half:tokens, :] = sel_hi
        w_out_ref[0:half, :] = w_lo
        w_out_ref[half:tokens, :] = w_hi
        var_ref[...] = jnp.reshape(var, (1, 1))


@jax.jit
def kernel(hidden_states, W):
    tokens, hidden = hidden_states.shape
    experts = W.shape[0]
    nblk = tokens // TOKEN_BLOCK

    logits, sel, wts, var, ent_sum = pl.pallas_call(
        _fused_body,
        grid=(nblk,),
        in_specs=[
            pl.BlockSpec((TOKEN_BLOCK, hidden), lambda j: (j, 0)),
            pl.BlockSpec((experts, hidden), lambda j: (0, 0)),
        ],
        out_specs=[
            pl.BlockSpec((TOKEN_BLOCK, experts), lambda j: (j, 0)),
            pl.BlockSpec((tokens, 1), lambda j: (0, 0)),
            pl.BlockSpec((tokens, 1), lambda j: (0, 0)),
            pl.BlockSpec((1, 1), lambda j: (0, 0)),
            pl.BlockSpec((1, 1), lambda j: (0, 0)),
        ],
        out_shape=[
            jax.ShapeDtypeStruct((tokens, experts), jnp.float32),
            jax.ShapeDtypeStruct((tokens, 1), jnp.int32),
            jax.ShapeDtypeStruct((tokens, 1), jnp.float32),
            jax.ShapeDtypeStruct((1, 1), jnp.float32),
            jax.ShapeDtypeStruct((1, 1), jnp.float32),
        ],
        scratch_shapes=[pltpu.VMEM((tokens // 2, 2 * experts), jnp.float32)],
    )(hidden_states, W)

    entropy = ent_sum[0, 0] / jnp.float32(tokens)
    return (logits, sel, wts, var[0, 0], entropy)


# speedup vs baseline: 7.2465x; 1.0098x over previous
import functools

import jax
import jax.numpy as jnp
from jax.experimental import pallas as pl
from jax.experimental.pallas import tpu as pltpu

TOKEN_BLOCK = 1024


def _dup(v):
    return jnp.concatenate([v, v], axis=1)


def _routing(pw, tokens, experts):
    half = tokens // 2
    k = min(tokens // experts, tokens)
    pb = jax.lax.bitcast_convert_type(pw, jnp.int32)

    def fold(c):
        return c[:, :experts] + c[:, experts:]

    tbits = jnp.zeros((1, experts), jnp.int32)
    for b in range(29, -1, -1):
        cand = tbits | jnp.int32(1 << b)
        cnt = fold(jnp.sum((pb >= _dup(cand)).astype(jnp.int32), axis=0,
                           keepdims=True))
        tbits = jnp.where(cnt >= k, cand, tbits)

    t128 = _dup(tbits)
    gt = pb > t128
    tie = pb == t128
    cnt_gt = fold(jnp.sum(gt.astype(jnp.int32), axis=0, keepdims=True))
    cnt_tie = fold(jnp.sum(tie.astype(jnp.int32), axis=0, keepdims=True))
    r = k - cnt_gt

    laneio = jax.lax.broadcasted_iota(jnp.int32, (half, 2 * experts), 1)
    hi_half = laneio >= experts
    idxw = (jax.lax.broadcasted_iota(jnp.int32, (half, 2 * experts), 0)
            + jnp.where(hi_half, half, 0))

    def tie_free(_):
        return jnp.full((1, experts), tokens, jnp.int32)

    def with_ties(_):
        istar = jnp.zeros((1, experts), jnp.int32)
        for b in range(12, -1, -1):
            cand = istar + jnp.int32(1 << b)
            c = fold(jnp.sum((tie & (idxw < _dup(cand))).astype(jnp.int32),
                             axis=0, keepdims=True))
            istar = jnp.where(c < r, cand, istar)
        return istar

    any_tie = jnp.max(cnt_gt + cnt_tie) > k
    istar = jax.lax.cond(any_tie, with_ties, tie_free, 0)

    sel_mask = gt | (tie & (idxw <= _dup(istar)))

    eio = jnp.where(hi_half, laneio - experts, laneio)
    neg = jnp.int32(-1)
    e_lo = jnp.max(jnp.where(sel_mask & ~hi_half, eio, neg), axis=1,
                   keepdims=True)
    e_hi = jnp.max(jnp.where(sel_mask & hi_half, eio, neg), axis=1,
                   keepdims=True)
    w_lo = jnp.max(jnp.where(sel_mask & ~hi_half & (eio == e_lo), pw, 0.0),
                   axis=1, keepdims=True)
    w_hi = jnp.max(jnp.where(sel_mask & hi_half & (eio == e_hi), pw, 0.0),
                   axis=1, keepdims=True)
    sel_lo = jnp.maximum(e_lo, 0)
    sel_hi = jnp.maximum(e_hi, 0)

    selw = jnp.where(hi_half, sel_hi, sel_lo)
    counts = fold(jnp.sum((selw == eio).astype(jnp.float32), axis=0,
                          keepdims=True).astype(jnp.int32)).astype(jnp.float32)
    load = counts / jnp.float32(tokens)
    mu = jnp.sum(load) / jnp.float32(experts)
    var = jnp.sum((load - mu) ** 2) / jnp.float32(experts - 1)
    return sel_lo, sel_hi, w_lo, w_hi, var


def _fused_body(x_ref, w_ref, logits_ref, sel_ref, w_out_ref, var_ref,
                ent_ref, probs_ref):
    j = pl.program_id(0)
    nblk = pl.num_programs(0)
    x = x_ref[...]
    w = w_ref[...]
    logits = jax.lax.dot_general(
        x, w, (((1,), (1,)), ((), ())), preferred_element_type=jnp.float32
    )
    logits_ref[...] = logits
    m = jnp.max(logits, axis=-1, keepdims=True)
    unnorm = jnp.exp(logits - m)
    p = unnorm / jnp.sum(unnorm, axis=-1, keepdims=True)
    blk = logits.shape[0]
    half_blocks = nblk // 2
    row = jnp.where(j < half_blocks, j, j - half_blocks) * blk

    @pl.when(j < half_blocks)
    def _():
        probs_ref[pl.ds(row, blk), 0:64] = p

    @pl.when(j >= half_blocks)
    def _():
        probs_ref[pl.ds(row, blk), 64:128] = p

    ent = -jnp.sum(p * jnp.log(p + 1e-8))

    @pl.when(j == 0)
    def _():
        ent_ref[...] = jnp.zeros_like(ent_ref)

    ent_ref[...] += jnp.reshape(ent, (1, 1))

    @pl.when(j == nblk - 1)
    def _():
        half, twoe = probs_ref.shape
        tokens = half * 2
        experts = twoe // 2
        sel_lo, sel_hi, w_lo, w_hi, var = _routing(
            probs_ref[...], tokens, experts)
        sel_ref[0:half, :] = sel_lo
        sel_ref[half:tokens, :] = sel_hi
        w_out_ref[0:half, :] = w_lo
        w_out_ref[half:tokens, :] = w_hi
        var_ref[...] = jnp.reshape(var, (1, 1))


@jax.jit
def kernel(hidden_states, W):
    tokens, hidden = hidden_states.shape
    experts = W.shape[0]
    nblk = tokens // TOKEN_BLOCK

    logits, sel, wts, var, ent_sum = pl.pallas_call(
        _fused_body,
        grid=(nblk,),
        in_specs=[
            pl.BlockSpec((TOKEN_BLOCK, hidden), lambda j: (j, 0)),
            pl.BlockSpec((experts, hidden), lambda j: (0, 0)),
        ],
        out_specs=[
            pl.BlockSpec((TOKEN_BLOCK, experts), lambda j: (j, 0)),
            pl.BlockSpec((tokens, 1), lambda j: (0, 0)),
            pl.BlockSpec((tokens, 1), lambda j: (0, 0)),
            pl.BlockSpec((1, 1), lambda j: (0, 0)),
            pl.BlockSpec((1, 1), lambda j: (0, 0)),
        ],
        out_shape=[
            jax.ShapeDtypeStruct((tokens, experts), jnp.float32),
            jax.ShapeDtypeStruct((tokens, 1), jnp.int32),
            jax.ShapeDtypeStruct((tokens, 1), jnp.float32),
            jax.ShapeDtypeStruct((1, 1), jnp.float32),
            jax.ShapeDtypeStruct((1, 1), jnp.float32),
        ],
        scratch_shapes=[pltpu.VMEM((tokens // 2, 2 * experts), jnp.float32)],
    )(hidden_states, W)

    entropy = ent_sum[0, 0] / jnp.float32(tokens)
    return (logits, sel, wts, var[0, 0], entropy)
